# baseline (speedup 1.0000x reference)
import functools

import jax
import jax.numpy as jnp
from jax import lax
from jax.experimental import pallas as pl
from jax.experimental.pallas import tpu as pltpu
from jax.experimental.pallas import tpu_sc as plsc


@functools.cache
def _make_broadcast_kernel(batch, seq_len, dim, dtype):
    info = plsc.get_sparse_core_info()
    num_workers = info.num_cores * info.num_subcores
    num_cores = info.num_cores
    assert seq_len % num_workers == 0
    rows_per_worker = seq_len // num_workers
    max_chunk = (131071 // dim) & ~7
    chunks = []
    left = rows_per_worker
    while left > 0:
        c = min(max_chunk, left)
        chunks.append(c)
        left -= c
    buf_rows = max(chunks)

    mesh = plsc.VectorSubcoreMesh(core_axis_name="c", subcore_axis_name="s")

    @functools.partial(
        pl.kernel,
        out_type=jax.ShapeDtypeStruct((batch, seq_len, dim), dtype),
        mesh=mesh,
        scratch_types=[
            pltpu.VMEM((buf_rows, dim), dtype),
            pltpu.SemaphoreType.DMA,
        ],
    )
    def bcast(w_hbm, out_hbm, buf, wsem):
        wid = lax.axis_index("s") * num_cores + lax.axis_index("c")
        base = wid * rows_per_worker
        off = 0
        for c in chunks:
            r0 = base + off
            pltpu.sync_copy(w_hbm.at[pl.ds(r0, c)], buf.at[pl.ds(0, c)])
            hs = [
                pltpu.async_copy(
                    buf.at[pl.ds(0, c)], out_hbm.at[b, pl.ds(r0, c)], wsem
                )
                for b in range(batch)
            ]
            for h in hs:
                h.wait()
            off += c

    return bcast


def kernel(x, weight):
    batch, seq_len, dim = x.shape
    return _make_broadcast_kernel(batch, seq_len, dim, weight.dtype)(weight)
